# baseline (speedup 1.0000x reference)
import jax
import jax.numpy as jnp
from jax import lax
from jax.experimental import pallas as pl
from jax.experimental.pallas import tpu as pltpu
from jax.experimental.pallas import tpu_sc as plsc

_N = 10000
_E = 320000
_D_IN = 128
_D_H = 128
_D_OUT = 64

_NC = 2
_NS = 16
_NW = _NC * _NS
_CHUNK = 128
_NCH = _E // _CHUNK
_NCHP = 2512
_CPW = 80
_CSTG = 16
_RPT = 632
_N_PAD = _NS * _RPT
_CNT_N = 10240
_CSLC = _CNT_N // _NS


def _make_sc_segsum(d, with_counts):
  mesh = plsc.VectorSubcoreMesh(
      core_axis_name="c", subcore_axis_name="s",
      num_cores=_NC, num_subcores=_NS)

  nbuf = 2 if d >= 128 else 6
  out_type = [jax.ShapeDtypeStruct((_NC * _N_PAD, d), jnp.float32)]
  scratch = (
      [pltpu.VMEM((_CSTG, _CHUNK), jnp.int32),
       pltpu.VMEM((_CSTG, _CHUNK), jnp.int32)]
      + [pltpu.VMEM((_CHUNK, d), jnp.float32)] * nbuf
      + [pltpu.VMEM_SHARED((_N_PAD, d), jnp.float32)]
      + [pltpu.SemaphoreType.DMA] * nbuf
  )
  if with_counts:
    out_type.append(jax.ShapeDtypeStruct((_NC, _CNT_N), jnp.float32))
    out_type.append(jax.ShapeDtypeStruct((_NW, _CNT_N), jnp.float32))
    scratch += [
        pltpu.VMEM((_CNT_N,), jnp.float32),
        pltpu.VMEM((_CSLC,), jnp.float32),
        pltpu.VMEM((_CSLC,), jnp.float32),
    ]

  def body(*refs):
    table, edges = refs[0:2]
    if with_counts:
      out_s, out_c, out_hist = refs[2:5]
      p = 5
    else:
      out_s = refs[2]
      p = 3
    src_v, dst_v = refs[p:p + 2]
    p += 2
    bufs = refs[p:p + nbuf]
    p += nbuf
    acc_sh = refs[p]
    p += 1
    sems = refs[p:p + nbuf]
    p += nbuf
    if with_counts:
      cnt_v, peer_v, comb_v = refs[p:p + 3]
    rows_a = bufs[0]

    c = lax.axis_index("c")
    s = lax.axis_index("s")
    wid = c * _NS + s
    r0 = s * _RPT

    def zrow(i, carry):
      for k in range(d // 16):
        rows_a[i, pl.ds(k * 16, 16)] = jnp.zeros((16,), jnp.float32)
      return carry
    lax.fori_loop(0, _CHUNK, zrow, 0)
    for k, sz in ((0, 128), (128, 128), (256, 128), (384, 128), (512, 120)):
      pltpu.sync_copy(rows_a.at[pl.ds(0, sz)], acc_sh.at[pl.ds(r0 + k, sz)])
    if with_counts:
      def zero_hist(i, carry):
        cnt_v[pl.ds(i * 16, 16)] = jnp.zeros((16,), jnp.float32)
        return carry
      lax.fori_loop(0, _CNT_N // 16, zero_hist, 0)
    plsc.subcore_barrier()

    def block(ph, carry):
      i0 = wid * _CPW + ph * _CSTG
      pltpu.sync_copy(edges.at[0, pl.ds(i0, _CSTG)], src_v)
      pltpu.sync_copy(edges.at[1, pl.ds(i0, _CSTG)], dst_v)
      descs = [None] * nbuf

      nb = len(bufs)

      def start(j):
        descs[j % nb] = pltpu.async_copy(
            table.at[src_v.at[j]], bufs[j % nb], sems[j % nb])

      def finish(j):
        if with_counts:
          for k in range(_CHUNK // 16):
            dd = dst_v[j, pl.ds(k * 16, 16)]
            cnt, last = plsc.scan_count(dd)
            plsc.addupdate_scatter(cnt_v, [dd], cnt.astype(jnp.float32),
                                   mask=last)
        descs[j % nb].wait()
        pltpu.sync_copy(bufs[j % nb], acc_sh.at[dst_v.at[j]], add=True)

      for j in range(nb - 1):
        start(j)
      for j in range(_CSTG):
        if j + nb - 1 < _CSTG:
          start(j + nb - 1)
        finish(j)
      return carry

    nblk = jnp.where(wid == _NW - 1, 2, _CPW // _CSTG)
    lax.fori_loop(0, nblk, block, 0)
    if with_counts:
      pltpu.async_copy(cnt_v, out_hist.at[wid], sems[0])
    plsc.subcore_barrier()

    o0 = c * _N_PAD + r0
    for k, sz in ((0, 128), (128, 128), (256, 128), (384, 128), (512, 120)):
      pltpu.sync_copy(acc_sh.at[pl.ds(r0 + k, sz)], rows_a.at[pl.ds(0, sz)])
      pltpu.sync_copy(rows_a.at[pl.ds(0, sz)], out_s.at[pl.ds(o0 + k, sz)])

    if with_counts:
      pltpu.make_async_copy(cnt_v, out_hist.at[wid], sems[0]).wait()
      plsc.subcore_barrier()
      b0 = s * _CSLC

      def zero_comb(i, carry):
        comb_v[pl.ds(i * 16, 16)] = jnp.zeros((16,), jnp.float32)
        return carry
      lax.fori_loop(0, _CSLC // 16, zero_comb, 0)
      for t in range(_NS):
        pltpu.sync_copy(out_hist.at[c * _NS + t, pl.ds(b0, _CSLC)], peer_v)

        def addup(i, carry):
          comb_v[pl.ds(i * 16, 16)] = (
              comb_v[pl.ds(i * 16, 16)] + peer_v[pl.ds(i * 16, 16)])
          return carry
        lax.fori_loop(0, _CSLC // 16, addup, 0)
      pltpu.sync_copy(comb_v, out_c.at[c, pl.ds(b0, _CSLC)])

  return pl.kernel(
      body, out_type=out_type, mesh=mesh, scratch_types=scratch,
      compiler_params=pltpu.CompilerParams(
          needs_layout_passes=False,
          use_tc_tiling_on_sc=(d % 128 == 0)))


_BLK = 2 * _RPT
_GRID = 8
_full = lambda shape: pl.BlockSpec(shape, lambda i: (0, 0))
_row = lambda width: pl.BlockSpec((_BLK, width), lambda i: (i, 0))
_rowb = lambda width: pl.BlockSpec((_BLK, width), lambda i: (i + _GRID, 0))


def _tc_dense1(x, sums1, cn, Wl1, Wr1, b1, Wl2, Wr2, b2):
  def tc_body(x_r, sa_r, sb_r, cn_r, wl1_r, wr1_r, b1_r,
              wl2_r, wr2_r, b2_r, p2_r, r2_r):
    cnt = jnp.maximum(cn_r[:, 0:1], 1.0)
    mean = (sa_r[...] + sb_r[...]) / cnt
    dn = (((1,), (1,)), ((), ()))
    h = lax.dot_general(mean, wl1_r[...], dn,
                        preferred_element_type=jnp.float32)
    h = h + b1_r[...] + lax.dot_general(x_r[...], wr1_r[...], dn,
                                        preferred_element_type=jnp.float32)
    h = jnp.maximum(h, 0.0)
    p2_r[...] = lax.dot_general(h, wl2_r[...], dn,
                                preferred_element_type=jnp.float32)
    r2_r[...] = lax.dot_general(h, wr2_r[...], dn,
                                preferred_element_type=jnp.float32) + b2_r[...]

  return pl.pallas_call(
      tc_body,
      grid=(_GRID,),
      in_specs=[_row(_D_IN), _row(_D_H), _rowb(_D_H), _row(8),
                _full((_D_H, _D_IN)), _full((_D_H, _D_IN)), _full((1, _D_H)),
                _full((_D_OUT, _D_H)), _full((_D_OUT, _D_H)),
                _full((1, _D_OUT))],
      out_specs=[_row(_D_OUT), _row(_D_OUT)],
      out_shape=[jax.ShapeDtypeStruct((_N, _D_OUT), jnp.float32),
                 jax.ShapeDtypeStruct((_N, _D_OUT), jnp.float32)],
  )(x, sums1, sums1, cn, Wl1, Wr1, b1, Wl2, Wr2, b2)


def _tc_dense2(sums2, cn, r2):
  def tc_body(sa_r, sb_r, cn_r, r2_r, o_r):
    cnt = jnp.maximum(cn_r[:, 0:1], 1.0)
    o = (sa_r[...] + sb_r[...]) / cnt + r2_r[...]
    o = o - jnp.max(o, axis=1, keepdims=True)
    lse = jnp.log(jnp.sum(jnp.exp(o), axis=1, keepdims=True))
    o_r[...] = o - lse

  return pl.pallas_call(
      tc_body,
      grid=(_GRID,),
      in_specs=[_row(_D_OUT), _rowb(_D_OUT), _row(8), _row(_D_OUT)],
      out_specs=_row(_D_OUT),
      out_shape=jax.ShapeDtypeStruct((_N, _D_OUT), jnp.float32),
  )(sums2, sums2, cn, r2)


@jax.jit
def kernel(x, edge_index, Wl1, Wr1, b1, Wl2, Wr2, b2):
  e3 = edge_index.reshape(2, _NCH, _CHUNK)
  pad_i = jnp.arange((_NCHP - _NCH) * _CHUNK,
                     dtype=jnp.int32).reshape(_NCHP - _NCH, _CHUNK)
  edges3 = jnp.concatenate(
      [e3, jnp.stack([pad_i % _CHUNK, _N + pad_i % (_N_PAD - _N)])], axis=1)

  sc1 = _make_sc_segsum(_D_H, with_counts=True)
  sums1, cnts, _hist = sc1(x, edges3)

  cn = jnp.broadcast_to((cnts[0] + cnts[1])[:_N, None], (_N, 8))

  p2, r2 = _tc_dense1(x, sums1, cn,
                      Wl1, Wr1, b1.reshape(1, _D_H),
                      Wl2, Wr2, b2.reshape(1, _D_OUT))

  sc2 = _make_sc_segsum(_D_OUT, with_counts=False)
  (sums2,) = sc2(p2, edges3)

  return _tc_dense2(sums2, cn, r2)


# speedup vs baseline: 1.0120x; 1.0120x over previous
import jax
import jax.numpy as jnp
from jax import lax
from jax.experimental import pallas as pl
from jax.experimental.pallas import tpu as pltpu
from jax.experimental.pallas import tpu_sc as plsc

_N = 10000
_E = 320000
_D_IN = 128
_D_H = 128
_D_OUT = 64

_NC = 2
_NS = 16
_NW = _NC * _NS
_CHUNK = 128
_NCH = _E // _CHUNK
_NCHP = 2512
_CPW = 80
_CSTG = 16
_RPT = 632
_N_PAD = _NS * _RPT
_CNT_N = 10240
_CSLC = _CNT_N // _NS


def _make_sc_segsum(d, with_counts):
  mesh = plsc.VectorSubcoreMesh(
      core_axis_name="c", subcore_axis_name="s",
      num_cores=_NC, num_subcores=_NS)

  nbuf = 2 if d >= 128 else 6
  out_type = [jax.ShapeDtypeStruct((_NC * _N_PAD, d), jnp.float32)]
  scratch = (
      [pltpu.VMEM((_CSTG, _CHUNK), jnp.int32),
       pltpu.VMEM((_CSTG, _CHUNK), jnp.int32)]
      + [pltpu.VMEM((_CHUNK, d), jnp.float32)] * nbuf
      + [pltpu.VMEM_SHARED((_N_PAD, d), jnp.float32)]
      + [pltpu.SemaphoreType.DMA] * nbuf
  )
  if with_counts:
    out_type.append(jax.ShapeDtypeStruct((_NC, _CNT_N), jnp.float32))
    out_type.append(jax.ShapeDtypeStruct((_NW, _CNT_N), jnp.float32))
    scratch += [
        pltpu.VMEM((_CNT_N,), jnp.float32),
        pltpu.VMEM((_CSLC,), jnp.float32),
        pltpu.VMEM((_CSLC,), jnp.float32),
    ]

  def body(*refs):
    table, edges = refs[0:2]
    if with_counts:
      out_s, out_c, out_hist = refs[2:5]
      p = 5
    else:
      out_s = refs[2]
      p = 3
    src_v, dst_v = refs[p:p + 2]
    p += 2
    bufs = refs[p:p + nbuf]
    p += nbuf
    acc_sh = refs[p]
    p += 1
    sems = refs[p:p + nbuf]
    p += nbuf
    if with_counts:
      cnt_v, peer_v, comb_v = refs[p:p + 3]
    rows_a = bufs[0]

    c = lax.axis_index("c")
    s = lax.axis_index("s")
    wid = c * _NS + s
    r0 = s * _RPT

    def zrow(i, carry):
      for k in range(d // 16):
        rows_a[i, pl.ds(k * 16, 16)] = jnp.zeros((16,), jnp.float32)
      return carry
    lax.fori_loop(0, _CHUNK, zrow, 0)
    for k, sz in ((0, 128), (128, 128), (256, 128), (384, 128), (512, 120)):
      pltpu.sync_copy(rows_a.at[pl.ds(0, sz)], acc_sh.at[pl.ds(r0 + k, sz)])
    if with_counts:
      def zero_hist(i, carry):
        cnt_v[pl.ds(i * 16, 16)] = jnp.zeros((16,), jnp.float32)
        return carry
      lax.fori_loop(0, _CNT_N // 16, zero_hist, 0)
    plsc.subcore_barrier()

    def block(ph, carry):
      i0 = wid * _CPW + ph * _CSTG
      pltpu.sync_copy(edges.at[0, pl.ds(i0, _CSTG)], src_v)
      pltpu.sync_copy(edges.at[1, pl.ds(i0, _CSTG)], dst_v)
      descs = [None] * nbuf

      nb = len(bufs)

      def start(j):
        descs[j % nb] = pltpu.async_copy(
            table.at[src_v.at[j]], bufs[j % nb], sems[j % nb])

      def finish(j):
        if with_counts:
          for k in range(_CHUNK // 16):
            dd = dst_v[j, pl.ds(k * 16, 16)]
            cnt, last = plsc.scan_count(dd)
            plsc.addupdate_scatter(cnt_v, [dd], cnt.astype(jnp.float32),
                                   mask=last)
        descs[j % nb].wait()
        pltpu.sync_copy(bufs[j % nb], acc_sh.at[dst_v.at[j]], add=True)

      for j in range(nb - 1):
        start(j)
      for j in range(_CSTG):
        if j + nb - 1 < _CSTG:
          start(j + nb - 1)
        finish(j)
      return carry

    nblk = jnp.where(wid == _NW - 1, 2, _CPW // _CSTG)
    lax.fori_loop(0, nblk, block, 0)
    if with_counts:
      pltpu.async_copy(cnt_v, out_hist.at[wid], sems[0])
    plsc.subcore_barrier()

    o0 = c * _N_PAD + r0
    for k, sz in ((0, 128), (128, 128), (256, 128), (384, 128), (512, 120)):
      pltpu.sync_copy(acc_sh.at[pl.ds(r0 + k, sz)], rows_a.at[pl.ds(0, sz)])
      pltpu.sync_copy(rows_a.at[pl.ds(0, sz)], out_s.at[pl.ds(o0 + k, sz)])

    if with_counts:
      pltpu.make_async_copy(cnt_v, out_hist.at[wid], sems[0]).wait()
      plsc.subcore_barrier()
      b0 = s * _CSLC

      def zero_comb(i, carry):
        comb_v[pl.ds(i * 16, 16)] = jnp.zeros((16,), jnp.float32)
        return carry
      lax.fori_loop(0, _CSLC // 16, zero_comb, 0)
      for t in range(_NS):
        pltpu.sync_copy(out_hist.at[c * _NS + t, pl.ds(b0, _CSLC)], peer_v)

        def addup(i, carry):
          comb_v[pl.ds(i * 16, 16)] = (
              comb_v[pl.ds(i * 16, 16)] + peer_v[pl.ds(i * 16, 16)])
          return carry
        lax.fori_loop(0, _CSLC // 16, addup, 0)
      pltpu.sync_copy(comb_v, out_c.at[c, pl.ds(b0, _CSLC)])

  return pl.kernel(
      body, out_type=out_type, mesh=mesh, scratch_types=scratch,
      compiler_params=pltpu.CompilerParams(
          needs_layout_passes=False,
          use_tc_tiling_on_sc=(d % 128 == 0)))


_BLK = 4 * _RPT
_GRID = 4
_full = lambda shape: pl.BlockSpec(shape, lambda i: (0, 0))
_row = lambda width: pl.BlockSpec((_BLK, width), lambda i: (i, 0))
_rowb = lambda width: pl.BlockSpec((_BLK, width), lambda i: (i + _GRID, 0))


def _tc_dense1(x, sums1, cn, Wl1, Wr1, b1, Wl2, Wr2, b2):
  def tc_body(x_r, sa_r, sb_r, cn_r, wl1_r, wr1_r, b1_r,
              wl2_r, wr2_r, b2_r, p2_r, r2_r):
    cnt = jnp.maximum(cn_r[:, 0:1], 1.0)
    mean = (sa_r[...] + sb_r[...]) / cnt
    dn = (((1,), (1,)), ((), ()))
    h = lax.dot_general(mean, wl1_r[...], dn,
                        preferred_element_type=jnp.float32)
    h = h + b1_r[...] + lax.dot_general(x_r[...], wr1_r[...], dn,
                                        preferred_element_type=jnp.float32)
    h = jnp.maximum(h, 0.0)
    p2_r[...] = lax.dot_general(h, wl2_r[...], dn,
                                preferred_element_type=jnp.float32)
    r2_r[...] = lax.dot_general(h, wr2_r[...], dn,
                                preferred_element_type=jnp.float32) + b2_r[...]

  return pl.pallas_call(
      tc_body,
      grid=(_GRID,),
      in_specs=[_row(_D_IN), _row(_D_H), _rowb(_D_H), _row(8),
                _full((_D_H, _D_IN)), _full((_D_H, _D_IN)), _full((1, _D_H)),
                _full((_D_OUT, _D_H)), _full((_D_OUT, _D_H)),
                _full((1, _D_OUT))],
      out_specs=[_row(_D_OUT), _row(_D_OUT)],
      out_shape=[jax.ShapeDtypeStruct((_N, _D_OUT), jnp.float32),
                 jax.ShapeDtypeStruct((_N, _D_OUT), jnp.float32)],
  )(x, sums1, sums1, cn, Wl1, Wr1, b1, Wl2, Wr2, b2)


def _tc_dense2(sums2, cn, r2):
  def tc_body(sa_r, sb_r, cn_r, r2_r, o_r):
    cnt = jnp.maximum(cn_r[:, 0:1], 1.0)
    o = (sa_r[...] + sb_r[...]) / cnt + r2_r[...]
    o = o - jnp.max(o, axis=1, keepdims=True)
    lse = jnp.log(jnp.sum(jnp.exp(o), axis=1, keepdims=True))
    o_r[...] = o - lse

  return pl.pallas_call(
      tc_body,
      grid=(_GRID,),
      in_specs=[_row(_D_OUT), _rowb(_D_OUT), _row(8), _row(_D_OUT)],
      out_specs=_row(_D_OUT),
      out_shape=jax.ShapeDtypeStruct((_N, _D_OUT), jnp.float32),
  )(sums2, sums2, cn, r2)


@jax.jit
def kernel(x, edge_index, Wl1, Wr1, b1, Wl2, Wr2, b2):
  e3 = edge_index.reshape(2, _NCH, _CHUNK)
  pad_i = jnp.arange((_NCHP - _NCH) * _CHUNK,
                     dtype=jnp.int32).reshape(_NCHP - _NCH, _CHUNK)
  edges3 = jnp.concatenate(
      [e3, jnp.stack([pad_i % _CHUNK, _N + pad_i % (_N_PAD - _N)])], axis=1)

  sc1 = _make_sc_segsum(_D_H, with_counts=True)
  sums1, cnts, _hist = sc1(x, edges3)

  cn = jnp.broadcast_to((cnts[0] + cnts[1])[:_N, None], (_N, 8))

  p2, r2 = _tc_dense1(x, sums1, cn,
                      Wl1, Wr1, b1.reshape(1, _D_H),
                      Wl2, Wr2, b2.reshape(1, _D_OUT))

  sc2 = _make_sc_segsum(_D_OUT, with_counts=False)
  (sums2,) = sc2(p2, edges3)

  return _tc_dense2(sums2, cn, r2)


# speedup vs baseline: 1.0180x; 1.0059x over previous
import jax
import jax.numpy as jnp
from jax import lax
from jax.experimental import pallas as pl
from jax.experimental.pallas import tpu as pltpu
from jax.experimental.pallas import tpu_sc as plsc

_N = 10000
_E = 320000
_D_IN = 128
_D_H = 128
_D_OUT = 64

_NC = 2
_NS = 16
_NW = _NC * _NS
_CHUNK = 128
_NCH = _E // _CHUNK
_NCHP = 2512
_CPW = 80
_CSTG = 16
_RPT = 632
_N_PAD = _NS * _RPT
_CNT_N = 10240
_CSLC = _CNT_N // _NS


def _make_sc_segsum(d, with_counts):
  mesh = plsc.VectorSubcoreMesh(
      core_axis_name="c", subcore_axis_name="s",
      num_cores=_NC, num_subcores=_NS)

  nbuf = 2 if d >= 128 else 6
  out_type = [jax.ShapeDtypeStruct((_NC * _N_PAD, d), jnp.float32)]
  scratch = (
      [pltpu.VMEM((_CSTG, _CHUNK), jnp.int32),
       pltpu.VMEM((_CSTG, _CHUNK), jnp.int32)]
      + [pltpu.VMEM((_CHUNK, d), jnp.float32)] * nbuf
      + [pltpu.VMEM_SHARED((_N_PAD, d), jnp.float32)]
      + [pltpu.SemaphoreType.DMA] * nbuf
  )
  if with_counts:
    out_type.append(jax.ShapeDtypeStruct((_NC, _CNT_N), jnp.float32))
    out_type.append(jax.ShapeDtypeStruct((_NW, _CNT_N), jnp.float32))
    scratch += [
        pltpu.VMEM((_CNT_N,), jnp.float32),
        pltpu.VMEM((_CSLC,), jnp.float32),
        pltpu.VMEM((_CSLC,), jnp.float32),
    ]

  def body(*refs):
    table, edges = refs[0:2]
    if with_counts:
      out_s, out_c, out_hist = refs[2:5]
      p = 5
    else:
      out_s = refs[2]
      p = 3
    src_v, dst_v = refs[p:p + 2]
    p += 2
    bufs = refs[p:p + nbuf]
    p += nbuf
    acc_sh = refs[p]
    p += 1
    sems = refs[p:p + nbuf]
    p += nbuf
    if with_counts:
      cnt_v, peer_v, comb_v = refs[p:p + 3]
    rows_a = bufs[0]

    c = lax.axis_index("c")
    s = lax.axis_index("s")
    wid = c * _NS + s
    r0 = s * _RPT

    def zrow(i, carry):
      for k in range(d // 16):
        rows_a[i, pl.ds(k * 16, 16)] = jnp.zeros((16,), jnp.float32)
      return carry
    lax.fori_loop(0, _CHUNK, zrow, 0)
    for k, sz in ((0, 128), (128, 128), (256, 128), (384, 128), (512, 120)):
      pltpu.sync_copy(rows_a.at[pl.ds(0, sz)], acc_sh.at[pl.ds(r0 + k, sz)])
    if with_counts:
      def zero_hist(i, carry):
        cnt_v[pl.ds(i * 16, 16)] = jnp.zeros((16,), jnp.float32)
        return carry
      lax.fori_loop(0, _CNT_N // 16, zero_hist, 0)
    plsc.subcore_barrier()

    def block(ph, carry):
      i0 = wid * _CPW + ph * _CSTG
      pltpu.sync_copy(edges.at[0, pl.ds(i0, _CSTG)], src_v)
      pltpu.sync_copy(edges.at[1, pl.ds(i0, _CSTG)], dst_v)
      descs = [None] * nbuf

      nb = len(bufs)

      def start(j):
        descs[j % nb] = pltpu.async_copy(
            table.at[src_v.at[j]], bufs[j % nb], sems[j % nb])

      def finish(j):
        if with_counts:
          for k in range(_CHUNK // 16):
            dd = dst_v[j, pl.ds(k * 16, 16)]
            cnt, last = plsc.scan_count(dd)
            plsc.addupdate_scatter(cnt_v, [dd], cnt.astype(jnp.float32),
                                   mask=last)
        descs[j % nb].wait()
        pltpu.sync_copy(bufs[j % nb], acc_sh.at[dst_v.at[j]], add=True)

      for j in range(nb - 1):
        start(j)
      for j in range(_CSTG):
        if j + nb - 1 < _CSTG:
          start(j + nb - 1)
        finish(j)
      return carry

    nblk = jnp.where(wid == _NW - 1, 2, _CPW // _CSTG)
    lax.fori_loop(0, nblk, block, 0)
    if with_counts:
      pltpu.async_copy(cnt_v, out_hist.at[wid], sems[0])
    plsc.subcore_barrier()

    o0 = c * _N_PAD + r0
    for k, sz in ((0, 128), (128, 128), (256, 128), (384, 128), (512, 120)):
      pltpu.sync_copy(acc_sh.at[pl.ds(r0 + k, sz)], rows_a.at[pl.ds(0, sz)])
      pltpu.sync_copy(rows_a.at[pl.ds(0, sz)], out_s.at[pl.ds(o0 + k, sz)])

    if with_counts:
      pltpu.make_async_copy(cnt_v, out_hist.at[wid], sems[0]).wait()
      plsc.subcore_barrier()
      b0 = s * _CSLC

      def zero_comb(i, carry):
        comb_v[pl.ds(i * 16, 16)] = jnp.zeros((16,), jnp.float32)
        return carry
      lax.fori_loop(0, _CSLC // 16, zero_comb, 0)
      for t in range(_NS):
        pltpu.sync_copy(out_hist.at[c * _NS + t, pl.ds(b0, _CSLC)], peer_v)

        def addup(i, carry):
          comb_v[pl.ds(i * 16, 16)] = (
              comb_v[pl.ds(i * 16, 16)] + peer_v[pl.ds(i * 16, 16)])
          return carry
        lax.fori_loop(0, _CSLC // 16, addup, 0)
      pltpu.sync_copy(comb_v, out_c.at[c, pl.ds(b0, _CSLC)])

  return pl.kernel(
      body, out_type=out_type, mesh=mesh, scratch_types=scratch,
      compiler_params=pltpu.CompilerParams(
          needs_layout_passes=False,
          use_tc_tiling_on_sc=(d % 128 == 0)))


_BLK = 8 * _RPT
_GRID = 2
_full = lambda shape: pl.BlockSpec(shape, lambda i: (0, 0))
_row = lambda width: pl.BlockSpec((_BLK, width), lambda i: (i, 0))
_rowb = lambda width: pl.BlockSpec((_BLK, width), lambda i: (i + _GRID, 0))


def _tc_dense1(x, sums1, cn, Wl1, Wr1, b1, Wl2, Wr2, b2):
  def tc_body(x_r, sa_r, sb_r, cn_r, wl1_r, wr1_r, b1_r,
              wl2_r, wr2_r, b2_r, p2_r, r2_r):
    cnt = jnp.maximum(cn_r[:, 0:1], 1.0)
    mean = (sa_r[...] + sb_r[...]) / cnt
    dn = (((1,), (1,)), ((), ()))
    h = lax.dot_general(mean, wl1_r[...], dn,
                        preferred_element_type=jnp.float32)
    h = h + b1_r[...] + lax.dot_general(x_r[...], wr1_r[...], dn,
                                        preferred_element_type=jnp.float32)
    h = jnp.maximum(h, 0.0)
    p2_r[...] = lax.dot_general(h, wl2_r[...], dn,
                                preferred_element_type=jnp.float32)
    r2_r[...] = lax.dot_general(h, wr2_r[...], dn,
                                preferred_element_type=jnp.float32) + b2_r[...]

  return pl.pallas_call(
      tc_body,
      grid=(_GRID,),
      in_specs=[_row(_D_IN), _row(_D_H), _rowb(_D_H), _row(8),
                _full((_D_H, _D_IN)), _full((_D_H, _D_IN)), _full((1, _D_H)),
                _full((_D_OUT, _D_H)), _full((_D_OUT, _D_H)),
                _full((1, _D_OUT))],
      out_specs=[_row(_D_OUT), _row(_D_OUT)],
      out_shape=[jax.ShapeDtypeStruct((_N, _D_OUT), jnp.float32),
                 jax.ShapeDtypeStruct((_N, _D_OUT), jnp.float32)],
  )(x, sums1, sums1, cn, Wl1, Wr1, b1, Wl2, Wr2, b2)


def _tc_dense2(sums2, cn, r2):
  def tc_body(sa_r, sb_r, cn_r, r2_r, o_r):
    cnt = jnp.maximum(cn_r[:, 0:1], 1.0)
    o = (sa_r[...] + sb_r[...]) / cnt + r2_r[...]
    o = o - jnp.max(o, axis=1, keepdims=True)
    lse = jnp.log(jnp.sum(jnp.exp(o), axis=1, keepdims=True))
    o_r[...] = o - lse

  return pl.pallas_call(
      tc_body,
      grid=(_GRID,),
      in_specs=[_row(_D_OUT), _rowb(_D_OUT), _row(8), _row(_D_OUT)],
      out_specs=_row(_D_OUT),
      out_shape=jax.ShapeDtypeStruct((_N, _D_OUT), jnp.float32),
  )(sums2, sums2, cn, r2)


@jax.jit
def kernel(x, edge_index, Wl1, Wr1, b1, Wl2, Wr2, b2):
  e3 = edge_index.reshape(2, _NCH, _CHUNK)
  pad_i = jnp.arange((_NCHP - _NCH) * _CHUNK,
                     dtype=jnp.int32).reshape(_NCHP - _NCH, _CHUNK)
  edges3 = jnp.concatenate(
      [e3, jnp.stack([pad_i % _CHUNK, _N + pad_i % (_N_PAD - _N)])], axis=1)

  sc1 = _make_sc_segsum(_D_H, with_counts=True)
  sums1, cnts, _hist = sc1(x, edges3)

  cn = jnp.broadcast_to((cnts[0] + cnts[1])[:_N, None], (_N, 8))

  p2, r2 = _tc_dense1(x, sums1, cn,
                      Wl1, Wr1, b1.reshape(1, _D_H),
                      Wl2, Wr2, b2.reshape(1, _D_OUT))

  sc2 = _make_sc_segsum(_D_OUT, with_counts=False)
  (sums2,) = sc2(p2, edges3)

  return _tc_dense2(sums2, cn, r2)


# speedup vs baseline: 1.0310x; 1.0127x over previous
import jax
import jax.numpy as jnp
from jax import lax
from jax.experimental import pallas as pl
from jax.experimental.pallas import tpu as pltpu
from jax.experimental.pallas import tpu_sc as plsc

_N = 10000
_E = 320000
_D_IN = 128
_D_H = 128
_D_OUT = 64

_NC = 2
_NS = 16
_NW = _NC * _NS
_CHUNK = 128
_NCH = _E // _CHUNK
_NCHP = 2512
_CPW = 80
_CSTG = 16
_RPT = 632
_N_PAD = _NS * _RPT
_CNT_N = 10240
_CSLC = _CNT_N // _NS


def _make_sc_segsum(d, with_counts):
  mesh = plsc.VectorSubcoreMesh(
      core_axis_name="c", subcore_axis_name="s",
      num_cores=_NC, num_subcores=_NS)

  nbuf = 2 if d >= 128 else 6
  out_type = [jax.ShapeDtypeStruct((_NC * _N_PAD, d), jnp.float32)]
  scratch = (
      [pltpu.VMEM((_CSTG, _CHUNK), jnp.int32),
       pltpu.VMEM((_CSTG, _CHUNK), jnp.int32)]
      + [pltpu.VMEM((_CHUNK, d), jnp.float32)] * nbuf
      + [pltpu.VMEM_SHARED((_N_PAD, d), jnp.float32)]
      + [pltpu.SemaphoreType.DMA] * nbuf
  )
  if with_counts:
    out_type.append(jax.ShapeDtypeStruct((_NC, _CNT_N), jnp.float32))
    out_type.append(jax.ShapeDtypeStruct((_NW, _CNT_N), jnp.float32))
    scratch += [
        pltpu.VMEM((_CNT_N,), jnp.float32),
        pltpu.VMEM((_CSLC,), jnp.float32),
        pltpu.VMEM((_CSLC,), jnp.float32),
        pltpu.SemaphoreType.DMA,
    ]

  def body(*refs):
    table, edges = refs[0:2]
    if with_counts:
      out_s, out_c, out_hist = refs[2:5]
      p = 5
    else:
      out_s = refs[2]
      p = 3
    src_v, dst_v = refs[p:p + 2]
    p += 2
    bufs = refs[p:p + nbuf]
    p += nbuf
    acc_sh = refs[p]
    p += 1
    sems = refs[p:p + nbuf]
    p += nbuf
    if with_counts:
      cnt_v, peer_v, comb_v, hist_sem = refs[p:p + 4]
    rows_a = bufs[0]

    c = lax.axis_index("c")
    s = lax.axis_index("s")
    wid = c * _NS + s
    r0 = s * _RPT

    def zrow(i, carry):
      for k in range(d // 16):
        rows_a[i, pl.ds(k * 16, 16)] = jnp.zeros((16,), jnp.float32)
      return carry
    lax.fori_loop(0, _CHUNK, zrow, 0)
    segs = ((0, 128), (128, 128), (256, 128), (384, 128), (512, 120))
    zds = [pltpu.async_copy(rows_a.at[pl.ds(0, sz)],
                            acc_sh.at[pl.ds(r0 + k, sz)], sems[0])
           for k, sz in segs]
    for zd in zds:
      zd.wait()
    if with_counts:
      def zero_hist(i, carry):
        cnt_v[pl.ds(i * 16, 16)] = jnp.zeros((16,), jnp.float32)
        return carry
      lax.fori_loop(0, _CNT_N // 16, zero_hist, 0)
    plsc.subcore_barrier()

    def block(ph, carry):
      i0 = wid * _CPW + ph * _CSTG
      pltpu.sync_copy(edges.at[0, pl.ds(i0, _CSTG)], src_v)
      pltpu.sync_copy(edges.at[1, pl.ds(i0, _CSTG)], dst_v)
      descs = [None] * nbuf

      nb = len(bufs)

      def start(j):
        descs[j % nb] = pltpu.async_copy(
            table.at[src_v.at[j]], bufs[j % nb], sems[j % nb])

      def finish(j):
        if with_counts:
          for k in range(_CHUNK // 16):
            dd = dst_v[j, pl.ds(k * 16, 16)]
            cnt, last = plsc.scan_count(dd)
            plsc.addupdate_scatter(cnt_v, [dd], cnt.astype(jnp.float32),
                                   mask=last)
        descs[j % nb].wait()
        pltpu.sync_copy(bufs[j % nb], acc_sh.at[dst_v.at[j]], add=True)

      for j in range(nb - 1):
        start(j)
      for j in range(_CSTG):
        if j + nb - 1 < _CSTG:
          start(j + nb - 1)
        finish(j)
      return carry

    nblk = jnp.where(wid == _NW - 1, 2, _CPW // _CSTG)
    lax.fori_loop(0, nblk, block, 0)
    if with_counts:
      hist_desc = pltpu.async_copy(cnt_v, out_hist.at[wid], hist_sem)
    plsc.subcore_barrier()

    o0 = c * _N_PAD + r0
    wds = [None, None]
    for i, (k, sz) in enumerate(segs):
      b = i & 1
      if wds[b] is not None:
        wds[b].wait()
      pltpu.sync_copy(acc_sh.at[pl.ds(r0 + k, sz)], bufs[b].at[pl.ds(0, sz)])
      wds[b] = pltpu.async_copy(bufs[b].at[pl.ds(0, sz)],
                                out_s.at[pl.ds(o0 + k, sz)], sems[b])
    for wd in wds:
      if wd is not None:
        wd.wait()

    if with_counts:
      hist_desc.wait()
      plsc.subcore_barrier()
      b0 = s * _CSLC

      def zero_comb(i, carry):
        comb_v[pl.ds(i * 16, 16)] = jnp.zeros((16,), jnp.float32)
        return carry
      lax.fori_loop(0, _CSLC // 16, zero_comb, 0)
      for t in range(_NS):
        pltpu.sync_copy(out_hist.at[c * _NS + t, pl.ds(b0, _CSLC)], peer_v)

        def addup(i, carry):
          comb_v[pl.ds(i * 16, 16)] = (
              comb_v[pl.ds(i * 16, 16)] + peer_v[pl.ds(i * 16, 16)])
          return carry
        lax.fori_loop(0, _CSLC // 16, addup, 0)
      pltpu.sync_copy(comb_v, out_c.at[c, pl.ds(b0, _CSLC)])

  return pl.kernel(
      body, out_type=out_type, mesh=mesh, scratch_types=scratch,
      compiler_params=pltpu.CompilerParams(
          needs_layout_passes=False,
          use_tc_tiling_on_sc=(d % 128 == 0)))


_BLK = 8 * _RPT
_GRID = 2
_full = lambda shape: pl.BlockSpec(shape, lambda i: (0, 0))
_row = lambda width: pl.BlockSpec((_BLK, width), lambda i: (i, 0))
_rowb = lambda width: pl.BlockSpec((_BLK, width), lambda i: (i + _GRID, 0))


def _tc_dense1(x, sums1, cn, Wl1, Wr1, b1, Wl2, Wr2, b2):
  def tc_body(x_r, sa_r, sb_r, cn_r, wl1_r, wr1_r, b1_r,
              wl2_r, wr2_r, b2_r, p2_r, r2_r):
    cnt = jnp.maximum(cn_r[:, 0:1], 1.0)
    mean = (sa_r[...] + sb_r[...]) / cnt
    dn = (((1,), (1,)), ((), ()))
    h = lax.dot_general(mean, wl1_r[...], dn,
                        preferred_element_type=jnp.float32)
    h = h + b1_r[...] + lax.dot_general(x_r[...], wr1_r[...], dn,
                                        preferred_element_type=jnp.float32)
    h = jnp.maximum(h, 0.0)
    p2_r[...] = lax.dot_general(h, wl2_r[...], dn,
                                preferred_element_type=jnp.float32)
    r2_r[...] = lax.dot_general(h, wr2_r[...], dn,
                                preferred_element_type=jnp.float32) + b2_r[...]

  return pl.pallas_call(
      tc_body,
      grid=(_GRID,),
      in_specs=[_row(_D_IN), _row(_D_H), _rowb(_D_H), _row(8),
                _full((_D_H, _D_IN)), _full((_D_H, _D_IN)), _full((1, _D_H)),
                _full((_D_OUT, _D_H)), _full((_D_OUT, _D_H)),
                _full((1, _D_OUT))],
      out_specs=[_row(_D_OUT), _row(_D_OUT)],
      out_shape=[jax.ShapeDtypeStruct((_N, _D_OUT), jnp.float32),
                 jax.ShapeDtypeStruct((_N, _D_OUT), jnp.float32)],
  )(x, sums1, sums1, cn, Wl1, Wr1, b1, Wl2, Wr2, b2)


def _tc_dense2(sums2, cn, r2):
  def tc_body(sa_r, sb_r, cn_r, r2_r, o_r):
    cnt = jnp.maximum(cn_r[:, 0:1], 1.0)
    o = (sa_r[...] + sb_r[...]) / cnt + r2_r[...]
    o = o - jnp.max(o, axis=1, keepdims=True)
    lse = jnp.log(jnp.sum(jnp.exp(o), axis=1, keepdims=True))
    o_r[...] = o - lse

  return pl.pallas_call(
      tc_body,
      grid=(_GRID,),
      in_specs=[_row(_D_OUT), _rowb(_D_OUT), _row(8), _row(_D_OUT)],
      out_specs=_row(_D_OUT),
      out_shape=jax.ShapeDtypeStruct((_N, _D_OUT), jnp.float32),
  )(sums2, sums2, cn, r2)


@jax.jit
def kernel(x, edge_index, Wl1, Wr1, b1, Wl2, Wr2, b2):
  e3 = edge_index.reshape(2, _NCH, _CHUNK)
  pad_i = jnp.arange((_NCHP - _NCH) * _CHUNK,
                     dtype=jnp.int32).reshape(_NCHP - _NCH, _CHUNK)
  edges3 = jnp.concatenate(
      [e3, jnp.stack([pad_i % _CHUNK, _N + pad_i % (_N_PAD - _N)])], axis=1)

  sc1 = _make_sc_segsum(_D_H, with_counts=True)
  sums1, cnts, _hist = sc1(x, edges3)

  cn = jnp.broadcast_to((cnts[0] + cnts[1])[:_N, None], (_N, 8))

  p2, r2 = _tc_dense1(x, sums1, cn,
                      Wl1, Wr1, b1.reshape(1, _D_H),
                      Wl2, Wr2, b2.reshape(1, _D_OUT))

  sc2 = _make_sc_segsum(_D_OUT, with_counts=False)
  (sums2,) = sc2(p2, edges3)

  return _tc_dense2(sums2, cn, r2)
